# baseline (speedup 1.0000x reference)
import jax
import jax.numpy as jnp
from jax import lax
from jax.experimental import pallas as pl
from jax.experimental.pallas import tpu as pltpu

_K = 4
_DN_T = (((1,), (1,)), ((), ()))


def _view(adj, deg, xv, xb, w0, w1, bv, wp):
    dis = jnp.where(deg > 0, jax.lax.rsqrt(deg), 0.0)
    y = (dis * xv).astype(jnp.float8_e4m3fn)
    z = jnp.dot(adj.astype(jnp.float8_e4m3fn), y,
                preferred_element_type=jnp.float32)
    tx1 = (-(dis * z)).astype(jnp.bfloat16)
    o = (lax.dot_general(xb, w0.astype(jnp.bfloat16), _DN_T,
                         preferred_element_type=jnp.float32)
         + lax.dot_general(tx1, w1.astype(jnp.bfloat16), _DN_T,
                           preferred_element_type=jnp.float32)
         + bv)
    return lax.dot_general(o.astype(jnp.bfloat16), wp.astype(jnp.bfloat16),
                           _DN_T, preferred_element_type=jnp.float32)


def _wait_and_scan(cps, buf, v, S):
    degs = []
    for k in range(_K):
        cps[v][k].wait()
        degs.append(jnp.sum(buf[v, k * S:(k + 1) * S, :], axis=1,
                            keepdims=True))
    return jnp.concatenate(degs, axis=0)


def _body(adj_hbm, x_ref, w0_ref, w1_ref, b_ref, wp_ref, bp_ref, out_ref,
          buf, sem):
    N = out_ref.shape[0]
    OUT = out_ref.shape[1]
    S = N // _K
    cps = [[pltpu.make_async_copy(adj_hbm.at[v, pl.ds(k * S, S), :],
                                  buf.at[v, pl.ds(k * S, S), :],
                                  sem.at[v, k])
            for k in range(_K)] for v in range(2)]
    for v in range(2):
        for k in range(_K):
            cps[v][k].start()
    xv = x_ref[...]
    xb = xv.astype(jnp.bfloat16)
    deg0 = _wait_and_scan(cps, buf, 0, S)
    acc = (bp_ref[...]
           + _view(buf[0], deg0, xv, xb, w0_ref[0], w1_ref[0], b_ref[0],
                   wp_ref[:, 0:OUT]))
    deg1 = _wait_and_scan(cps, buf, 1, S)
    out_ref[...] = acc + _view(buf[1], deg1, xv, xb, w0_ref[1], w1_ref[1],
                               b_ref[1], wp_ref[:, OUT:2 * OUT])


def kernel(x, adj_list, W0, W1, b, Wp, bp):
    B, N, C = x.shape
    V = adj_list.shape[0]
    OUT = W0.shape[1]
    x2 = x.reshape(N, C)
    b3 = b.reshape(V, 1, OUT)
    bp2 = bp.reshape(1, OUT)

    out = pl.pallas_call(
        _body,
        in_specs=[
            pl.BlockSpec(memory_space=pltpu.MemorySpace.HBM),
            pl.BlockSpec((N, C), lambda: (0, 0)),
            pl.BlockSpec((V, OUT, C), lambda: (0, 0, 0)),
            pl.BlockSpec((V, OUT, C), lambda: (0, 0, 0)),
            pl.BlockSpec((V, 1, OUT), lambda: (0, 0, 0)),
            pl.BlockSpec((OUT, V * OUT), lambda: (0, 0)),
            pl.BlockSpec((1, OUT), lambda: (0, 0)),
        ],
        out_specs=pl.BlockSpec((N, OUT), lambda: (0, 0)),
        out_shape=jax.ShapeDtypeStruct((N, OUT), jnp.float32),
        scratch_shapes=[
            pltpu.VMEM((V, N, N), jnp.float32),
            pltpu.SemaphoreType.DMA((V, _K)),
        ],
    )(adj_list, x2, W0, W1, b3, Wp, bp2)
    return out.reshape(B, N, OUT)


# speedup vs baseline: 1.1133x; 1.1133x over previous
import jax
import jax.numpy as jnp
from jax import lax
from jax.experimental import pallas as pl
from jax.experimental.pallas import tpu as pltpu

_DN_T = (((1,), (1,)), ((), ()))


def _view(adj, o_pre, xv, w1, wp):
    deg = jnp.sum(adj, axis=1, keepdims=True)
    dis = jnp.where(deg > 0, jax.lax.rsqrt(deg), 0.0)
    y = (dis * xv).astype(jnp.float8_e4m3fn)
    z = jnp.dot(adj.astype(jnp.float8_e4m3fn), y,
                preferred_element_type=jnp.float32)
    tx1 = (-(dis * z)).astype(jnp.bfloat16)
    o = o_pre + lax.dot_general(tx1, w1.astype(jnp.bfloat16), _DN_T,
                                preferred_element_type=jnp.float32)
    return lax.dot_general(o.astype(jnp.bfloat16), wp.astype(jnp.bfloat16),
                           _DN_T, preferred_element_type=jnp.float32)


def _body(adj_hbm, x_ref, w0_ref, w1_ref, b_ref, wp_ref, bp_ref, out_ref,
          buf, sem):
    OUT = out_ref.shape[1]
    cp0 = pltpu.make_async_copy(adj_hbm.at[0], buf.at[0], sem.at[0])
    cp1 = pltpu.make_async_copy(adj_hbm.at[1], buf.at[1], sem.at[1])
    cp0.start()
    cp1.start()
    xv = x_ref[...]
    xb = xv.astype(jnp.bfloat16)
    w0r = w0_ref[...].reshape(2 * OUT, w0_ref.shape[2])
    xw = lax.dot_general(xb, w0r.astype(jnp.bfloat16), _DN_T,
                         preferred_element_type=jnp.float32)
    o_pre0 = xw[:, 0:OUT] + b_ref[0]
    o_pre1 = xw[:, OUT:2 * OUT] + b_ref[1]
    cp0.wait()
    acc = (bp_ref[...]
           + _view(buf[0], o_pre0, xv, w1_ref[0], wp_ref[:, 0:OUT]))
    cp1.wait()
    out_ref[...] = acc + _view(buf[1], o_pre1, xv, w1_ref[1],
                               wp_ref[:, OUT:2 * OUT])


def kernel(x, adj_list, W0, W1, b, Wp, bp):
    B, N, C = x.shape
    V = adj_list.shape[0]
    OUT = W0.shape[1]
    x2 = x.reshape(N, C)
    b3 = b.reshape(V, 1, OUT)
    bp2 = bp.reshape(1, OUT)

    out = pl.pallas_call(
        _body,
        in_specs=[
            pl.BlockSpec(memory_space=pltpu.MemorySpace.HBM),
            pl.BlockSpec((N, C), lambda: (0, 0)),
            pl.BlockSpec((V, OUT, C), lambda: (0, 0, 0)),
            pl.BlockSpec((V, OUT, C), lambda: (0, 0, 0)),
            pl.BlockSpec((V, 1, OUT), lambda: (0, 0, 0)),
            pl.BlockSpec((OUT, V * OUT), lambda: (0, 0)),
            pl.BlockSpec((1, OUT), lambda: (0, 0)),
        ],
        out_specs=pl.BlockSpec((N, OUT), lambda: (0, 0)),
        out_shape=jax.ShapeDtypeStruct((N, OUT), jnp.float32),
        scratch_shapes=[
            pltpu.VMEM((V, N, N), jnp.float32),
            pltpu.SemaphoreType.DMA((V,)),
        ],
    )(adj_list, x2, W0, W1, b3, Wp, bp2)
    return out.reshape(B, N, OUT)
